# baseline (speedup 1.0000x reference)
import functools

import jax
import jax.numpy as jnp
from jax import lax
from jax.experimental import pallas as pl
from jax.experimental.pallas import tpu as pltpu
from jax.experimental.pallas import tpu_sc as plsc

VOCAB = 1000000
D = 64
DP = 128
B, L = 4096, 200
N_TOK = B * L
NW = 32
TPW = N_TOK // NW
SEQ_PER_W = TPW // L
SPB = 2
NBLK = SEQ_PER_W // SPB
CH = 128


VC = 2048


def _tp_body(in_ref, out_ref):
    i = lax.broadcasted_iota(jnp.int32, (D, D), 0)
    j = lax.broadcasted_iota(jnp.int32, (D, D), 1)
    eye = jnp.where(i == j, 1.0, 0.0).astype(jnp.float32)
    out_ref[:, 0:D] = lax.dot_general(
        in_ref[...], eye, (((0,), (0,)), ((), ())),
        precision=lax.Precision.HIGHEST)


def _transpose_pad_table(table_t):
    return pl.pallas_call(
        _tp_body,
        out_shape=jax.ShapeDtypeStruct((VOCAB, DP), jnp.float32),
        grid=(pl.cdiv(VOCAB, VC),),
        in_specs=[pl.BlockSpec((D, VC), lambda i: (0, i))],
        out_specs=pl.BlockSpec((VC, DP), lambda i: (i, 0)),
    )(table_t)


def _embed_lookup(targets_flat, table_padded):
    mesh = plsc.VectorSubcoreMesh(core_axis_name="c", subcore_axis_name="s")

    @functools.partial(
        pl.kernel,
        mesh=mesh,
        compiler_params=pltpu.CompilerParams(use_tc_tiling_on_sc=False),
        out_type=jax.ShapeDtypeStruct((B, L, DP), jnp.float32),
        scratch_types=[
            pltpu.VMEM((TPW,), jnp.int32),
            pltpu.VMEM((2, SPB, L, DP), jnp.float32),
            pltpu.SemaphoreType.DMA,
            pltpu.SemaphoreType.DMA,
            pltpu.SemaphoreType.DMA,
            pltpu.SemaphoreType.DMA,
        ],
    )
    def k(tgt_hbm, table_hbm, out_hbm, raw_v, rows_v,
          gsem0, gsem1, ssem0, ssem1):
        gsem = (gsem0, gsem1)
        ssem = (ssem0, ssem1)
        wid = lax.axis_index("s") * 2 + lax.axis_index("c")
        base = wid * TPW
        pltpu.sync_copy(tgt_hbm.at[pl.ds(base, TPW)], raw_v)
        for b in range(2):
            for s in range(SPB):
                pltpu.sync_copy(table_hbm.at[pl.ds(0, 8)],
                                rows_v.at[b, s, pl.ds(0, 8)])

        def fire_gather(i, b):
            for s in range(SPB):
                tok0 = (i * SPB + s) * L
                pltpu.async_copy(
                    table_hbm.at[raw_v.at[pl.ds(tok0, CH)]],
                    rows_v.at[b, s, pl.ds(1, CH)],
                    gsem[b])
                pltpu.async_copy(
                    table_hbm.at[raw_v.at[pl.ds(tok0 + CH, L - 1 - CH)]],
                    rows_v.at[b, s, pl.ds(1 + CH, L - 1 - CH)],
                    gsem[b])

        def wait_gather(b):
            for s in range(SPB):
                pltpu.make_async_copy(
                    table_hbm.at[raw_v.at[pl.ds(s * L, CH)]],
                    rows_v.at[b, s, pl.ds(1, CH)],
                    gsem[b]).wait()
                pltpu.make_async_copy(
                    table_hbm.at[raw_v.at[pl.ds(s * L + CH, L - 1 - CH)]],
                    rows_v.at[b, s, pl.ds(1 + CH, L - 1 - CH)],
                    gsem[b]).wait()

        def start_store(i, b):
            pltpu.async_copy(
                rows_v.at[b],
                out_hbm.at[pl.ds(wid * SEQ_PER_W + i * SPB, SPB)],
                ssem[b])

        def wait_store(b):
            pltpu.make_async_copy(
                rows_v.at[b], out_hbm.at[pl.ds(0, SPB)], ssem[b]).wait()

        fire_gather(0, 0)

        def blk2(g, carry):
            for b in range(2):
                i = g * 2 + b
                wait_gather(b)
                start_store(i, b)

                @pl.when(i + 1 < NBLK)
                def _():
                    @pl.when(i >= 1)
                    def _():
                        wait_store(1 - b)
                    fire_gather(i + 1, 1 - b)
            return carry

        lax.fori_loop(0, NBLK // 2, blk2, 0)
        wait_store(0)
        wait_store(1)

    return k(targets_flat, table_padded)


def kernel(targets, table):
    flat = targets.astype(jnp.int32).reshape(N_TOK)
    table_padded = _transpose_pad_table(table.T)
    out_padded = _embed_lookup(flat, table_padded)
    return out_padded[:, :, :D]


# speedup vs baseline: 1.0917x; 1.0917x over previous
import functools

import jax
import jax.numpy as jnp
from jax import lax
from jax.experimental import pallas as pl
from jax.experimental.pallas import tpu as pltpu
from jax.experimental.pallas import tpu_sc as plsc

VOCAB = 1000000
D = 64
DP = 128
B, L = 4096, 200
N_TOK = B * L
NW = 32
TPW = N_TOK // NW
SEQ_PER_W = TPW // L
SPB = 2
NBLK = SEQ_PER_W // SPB
CH = 128


VC = 2048


def _tp_body(in_ref, out_ref):
    i = lax.broadcasted_iota(jnp.int32, (D, D), 0)
    j = lax.broadcasted_iota(jnp.int32, (D, D), 1)
    eye = jnp.where(i == j, 1.0, 0.0).astype(jnp.float32)
    out_ref[:, 0:D] = lax.dot_general(
        in_ref[...], eye, (((0,), (0,)), ((), ())),
        precision=lax.Precision.HIGHEST)


def _transpose_pad_table(table_t):
    return pl.pallas_call(
        _tp_body,
        out_shape=jax.ShapeDtypeStruct((VOCAB, DP), jnp.float32),
        grid=(pl.cdiv(VOCAB, VC),),
        in_specs=[pl.BlockSpec((D, VC), lambda i: (0, i))],
        out_specs=pl.BlockSpec((VC, DP), lambda i: (i, 0)),
    )(table_t)


def _embed_lookup(targets_flat, table_padded):
    mesh = plsc.VectorSubcoreMesh(core_axis_name="c", subcore_axis_name="s")

    @functools.partial(
        pl.kernel,
        mesh=mesh,
        compiler_params=pltpu.CompilerParams(use_tc_tiling_on_sc=False),
        out_type=jax.ShapeDtypeStruct((B, L, DP), jnp.float32),
        scratch_types=[
            pltpu.VMEM((TPW,), jnp.int32),
            pltpu.VMEM((2, SPB, L, DP), jnp.float32),
            pltpu.SemaphoreType.DMA,
            pltpu.SemaphoreType.DMA,
            pltpu.SemaphoreType.DMA,
            pltpu.SemaphoreType.DMA,
        ],
    )
    def k(tgt_hbm, table_hbm, out_hbm, raw_v, rows_v,
          gsem0, gsem1, ssem0, ssem1):
        gsem = (gsem0, gsem1)
        ssem = (ssem0, ssem1)
        wid = lax.axis_index("s") * 2 + lax.axis_index("c")
        base = wid * TPW
        pltpu.sync_copy(tgt_hbm.at[pl.ds(base, TPW)], raw_v)
        for b in range(2):
            for s in range(SPB):
                pltpu.sync_copy(table_hbm.at[pl.ds(0, 8)],
                                rows_v.at[b, s, pl.ds(0, 8)])

        def fire_gather(i, b):
            for s in range(SPB):
                tok0 = (i * SPB + s) * L
                pltpu.async_copy(
                    table_hbm.at[raw_v.at[pl.ds(tok0, CH)]],
                    rows_v.at[b, s, pl.ds(1, CH)],
                    gsem[b])
                pltpu.async_copy(
                    table_hbm.at[raw_v.at[pl.ds(tok0 + CH, L - 1 - CH)]],
                    rows_v.at[b, s, pl.ds(1 + CH, L - 1 - CH)],
                    gsem[b])

        def wait_gather(b):
            for s in range(SPB):
                pltpu.make_async_copy(
                    table_hbm.at[raw_v.at[pl.ds(s * L, CH)]],
                    rows_v.at[b, s, pl.ds(1, CH)],
                    gsem[b]).wait()
                pltpu.make_async_copy(
                    table_hbm.at[raw_v.at[pl.ds(s * L + CH, L - 1 - CH)]],
                    rows_v.at[b, s, pl.ds(1 + CH, L - 1 - CH)],
                    gsem[b]).wait()

        def start_store(i, b):
            pltpu.async_copy(
                rows_v.at[b],
                out_hbm.at[pl.ds(wid * SEQ_PER_W + i * SPB, SPB)],
                ssem[b])

        def wait_store(b):
            pltpu.make_async_copy(
                rows_v.at[b], out_hbm.at[pl.ds(0, SPB)], ssem[b]).wait()

        fire_gather(0, 0)

        def blk2(g, carry):
            for b in range(2):
                i = g * 2 + b
                wait_gather(b)
                start_store(i, b)

                @pl.when(i + 1 < NBLK)
                def _():
                    @pl.when(i >= 1)
                    def _():
                        wait_store(1 - b)
                    fire_gather(i + 1, 1 - b)
            return carry

        lax.fori_loop(0, NBLK // 2, blk2, 0)
        wait_store(0)
        wait_store(1)

    return k(targets_flat, table_padded)


def kernel(targets, table):
    flat = targets.astype(jnp.int32).reshape(N_TOK)
    table_padded = jnp.pad(table, ((0, 0), (0, DP - D)))
    out_padded = _embed_lookup(flat, table_padded)
    return out_padded[:, :, :D]


# speedup vs baseline: 1.0921x; 1.0004x over previous
import functools

import jax
import jax.numpy as jnp
from jax import lax
from jax.experimental import pallas as pl
from jax.experimental.pallas import tpu as pltpu
from jax.experimental.pallas import tpu_sc as plsc

VOCAB = 1000000
D = 64
DP = 128
B, L = 4096, 200
N_TOK = B * L
NW = 32
TPW = N_TOK // NW
SEQ_PER_W = TPW // L
SPB = 2
NBLK = SEQ_PER_W // SPB
CH = 128


VC = 2048


def _tp_body(in_ref, out_ref):
    i = lax.broadcasted_iota(jnp.int32, (D, D), 0)
    j = lax.broadcasted_iota(jnp.int32, (D, D), 1)
    eye = jnp.where(i == j, 1.0, 0.0).astype(jnp.float32)
    out_ref[:, 0:D] = lax.dot_general(
        in_ref[...], eye, (((0,), (0,)), ((), ())),
        precision=lax.Precision.HIGHEST)


def _transpose_pad_table(table_t):
    return pl.pallas_call(
        _tp_body,
        out_shape=jax.ShapeDtypeStruct((VOCAB, DP), jnp.float32),
        grid=(pl.cdiv(VOCAB, VC),),
        in_specs=[pl.BlockSpec((D, VC), lambda i: (0, i))],
        out_specs=pl.BlockSpec((VC, DP), lambda i: (i, 0)),
    )(table_t)


NBUF = 4


def _embed_lookup(targets_flat, table_padded):
    mesh = plsc.VectorSubcoreMesh(core_axis_name="c", subcore_axis_name="s")

    @functools.partial(
        pl.kernel,
        mesh=mesh,
        compiler_params=pltpu.CompilerParams(use_tc_tiling_on_sc=False),
        out_type=jax.ShapeDtypeStruct((B, L, DP), jnp.float32),
        scratch_types=[
            pltpu.VMEM((TPW,), jnp.int32),
            pltpu.VMEM((NBUF, L, DP), jnp.float32),
            [pltpu.SemaphoreType.DMA] * NBUF,
            [pltpu.SemaphoreType.DMA] * NBUF,
        ],
    )
    def k(tgt_hbm, table_hbm, out_hbm, raw_v, rows_v, gsem, ssem):
        wid = lax.axis_index("s") * 2 + lax.axis_index("c")
        base = wid * TPW
        pltpu.sync_copy(tgt_hbm.at[pl.ds(base, TPW)], raw_v)
        for b in range(NBUF):
            pltpu.sync_copy(table_hbm.at[pl.ds(0, 8)],
                            rows_v.at[b, pl.ds(0, 8)])

        def fire_gather(i, b):
            tok0 = i * L
            pltpu.async_copy(
                table_hbm.at[raw_v.at[pl.ds(tok0, CH)]],
                rows_v.at[b, pl.ds(1, CH)],
                gsem[b])
            pltpu.async_copy(
                table_hbm.at[raw_v.at[pl.ds(tok0 + CH, L - 1 - CH)]],
                rows_v.at[b, pl.ds(1 + CH, L - 1 - CH)],
                gsem[b])

        def wait_gather(b):
            pltpu.make_async_copy(
                table_hbm.at[raw_v.at[pl.ds(0, CH)]],
                rows_v.at[b, pl.ds(1, CH)],
                gsem[b]).wait()
            pltpu.make_async_copy(
                table_hbm.at[raw_v.at[pl.ds(CH, L - 1 - CH)]],
                rows_v.at[b, pl.ds(1 + CH, L - 1 - CH)],
                gsem[b]).wait()

        def start_store(i, b):
            pltpu.async_copy(
                rows_v.at[b], out_hbm.at[wid * SEQ_PER_W + i], ssem[b])

        def wait_store(b):
            pltpu.make_async_copy(
                rows_v.at[b], out_hbm.at[0], ssem[b]).wait()

        for b in range(NBUF - 1):
            fire_gather(b, b)

        def blk(g, carry):
            for b in range(NBUF):
                i = g * NBUF + b
                wait_gather(b)
                start_store(i, b)

                bb = (b + NBUF - 1) % NBUF

                @pl.when(i + NBUF - 1 < SEQ_PER_W)
                def _():
                    @pl.when(i >= 1)
                    def _():
                        wait_store(bb)
                    fire_gather(i + NBUF - 1, bb)
            return carry

        lax.fori_loop(0, SEQ_PER_W // NBUF, blk, 0)
        for b in range(NBUF):
            wait_store(b)

    return k(targets_flat, table_padded)


def kernel(targets, table):
    flat = targets.astype(jnp.int32).reshape(N_TOK)
    table_padded = jnp.pad(table, ((0, 0), (0, DP - D)))
    out_padded = _embed_lookup(flat, table_padded)
    return out_padded[:, :, :D]


# speedup vs baseline: 1.1627x; 1.0646x over previous
import functools

import jax
import jax.numpy as jnp
from jax import lax
from jax.experimental import pallas as pl
from jax.experimental.pallas import tpu as pltpu
from jax.experimental.pallas import tpu_sc as plsc

VOCAB = 1000000
D = 64
DP = 128
B, L = 4096, 200
N_TOK = B * L
NW = 32
TPW = N_TOK // NW
SEQ_PER_W = TPW // L
SPB = 2
NBLK = SEQ_PER_W // SPB
CH = 128


VC = 2048


def _tp_body(in_ref, out_ref):
    out_ref[:, 0:D] = in_ref[...].T


def _transpose_pad_table(table_t):
    return pl.pallas_call(
        _tp_body,
        out_shape=jax.ShapeDtypeStruct((VOCAB, DP), jnp.float32),
        grid=(pl.cdiv(VOCAB, VC),),
        in_specs=[pl.BlockSpec((D, VC), lambda i: (0, i))],
        out_specs=pl.BlockSpec((VC, DP), lambda i: (i, 0)),
    )(table_t)


NBUF = 4


def _embed_lookup(targets_flat, table_padded):
    mesh = plsc.VectorSubcoreMesh(core_axis_name="c", subcore_axis_name="s")

    @functools.partial(
        pl.kernel,
        mesh=mesh,
        compiler_params=pltpu.CompilerParams(use_tc_tiling_on_sc=False),
        out_type=jax.ShapeDtypeStruct((B, L, DP), jnp.float32),
        scratch_types=[
            pltpu.VMEM((TPW,), jnp.int32),
            pltpu.VMEM((NBUF, L, DP), jnp.float32),
            [pltpu.SemaphoreType.DMA] * NBUF,
            [pltpu.SemaphoreType.DMA] * NBUF,
        ],
    )
    def k(tgt_hbm, table_hbm, out_hbm, raw_v, rows_v, gsem, ssem):
        wid = lax.axis_index("s") * 2 + lax.axis_index("c")
        base = wid * TPW
        pltpu.sync_copy(tgt_hbm.at[pl.ds(base, TPW)], raw_v)
        for b in range(NBUF):
            pltpu.sync_copy(table_hbm.at[pl.ds(0, 8)],
                            rows_v.at[b, pl.ds(0, 8)])

        def fire_gather(i, b):
            tok0 = i * L
            pltpu.async_copy(
                table_hbm.at[raw_v.at[pl.ds(tok0, CH)]],
                rows_v.at[b, pl.ds(1, CH)],
                gsem[b])
            pltpu.async_copy(
                table_hbm.at[raw_v.at[pl.ds(tok0 + CH, L - 1 - CH)]],
                rows_v.at[b, pl.ds(1 + CH, L - 1 - CH)],
                gsem[b])

        def wait_gather(b):
            pltpu.make_async_copy(
                table_hbm.at[raw_v.at[pl.ds(0, CH)]],
                rows_v.at[b, pl.ds(1, CH)],
                gsem[b]).wait()
            pltpu.make_async_copy(
                table_hbm.at[raw_v.at[pl.ds(CH, L - 1 - CH)]],
                rows_v.at[b, pl.ds(1 + CH, L - 1 - CH)],
                gsem[b]).wait()

        def start_store(i, b):
            pltpu.async_copy(
                rows_v.at[b], out_hbm.at[wid * SEQ_PER_W + i], ssem[b])

        def wait_store(b):
            pltpu.make_async_copy(
                rows_v.at[b], out_hbm.at[0], ssem[b]).wait()

        for b in range(NBUF - 1):
            fire_gather(b, b)

        def blk(g, carry):
            for b in range(NBUF):
                i = g * NBUF + b
                wait_gather(b)
                start_store(i, b)

                bb = (b + NBUF - 1) % NBUF

                @pl.when(i + NBUF - 1 < SEQ_PER_W)
                def _():
                    @pl.when(i >= 1)
                    def _():
                        wait_store(bb)
                    fire_gather(i + NBUF - 1, bb)
            return carry

        lax.fori_loop(0, SEQ_PER_W // NBUF, blk, 0)
        for b in range(NBUF):
            wait_store(b)

    return k(targets_flat, table_padded)


def kernel(targets, table):
    flat = targets.astype(jnp.int32).reshape(N_TOK)
    table_padded = _transpose_pad_table(table.T)
    out_padded = _embed_lookup(flat, table_padded)
    return out_padded[:, :, :D]


# speedup vs baseline: 1.4629x; 1.2582x over previous
import functools

import jax
import jax.numpy as jnp
from jax import lax
from jax.experimental import pallas as pl
from jax.experimental.pallas import tpu as pltpu
from jax.experimental.pallas import tpu_sc as plsc

VOCAB = 1000000
D = 64
DP = 128
B, L = 4096, 200
N_TOK = B * L
NW = 32
TPW = N_TOK // NW
SEQ_PER_W = TPW // L
SPB = 2
NBLK = SEQ_PER_W // SPB
CH = 128


VC = 8192


def _tp_body(in_ref, out_ref):
    out_ref[:, 0:D] = in_ref[...].T


def _transpose_pad_table(table_t):
    return pl.pallas_call(
        _tp_body,
        out_shape=jax.ShapeDtypeStruct((VOCAB, DP), jnp.float32),
        grid=(pl.cdiv(VOCAB, VC),),
        in_specs=[pl.BlockSpec((D, VC), lambda i: (0, i))],
        out_specs=pl.BlockSpec((VC, DP), lambda i: (i, 0)),
    )(table_t)


NBUF = 4


def _embed_lookup(targets_flat, table_padded):
    mesh = plsc.VectorSubcoreMesh(core_axis_name="c", subcore_axis_name="s")

    @functools.partial(
        pl.kernel,
        mesh=mesh,
        compiler_params=pltpu.CompilerParams(use_tc_tiling_on_sc=False),
        out_type=jax.ShapeDtypeStruct((B, L, DP), jnp.float32),
        scratch_types=[
            pltpu.VMEM((TPW,), jnp.int32),
            pltpu.VMEM((NBUF, L, DP), jnp.float32),
            [pltpu.SemaphoreType.DMA] * NBUF,
            [pltpu.SemaphoreType.DMA] * NBUF,
        ],
    )
    def k(tgt_hbm, table_hbm, out_hbm, raw_v, rows_v, gsem, ssem):
        wid = lax.axis_index("s") * 2 + lax.axis_index("c")
        base = wid * TPW
        pltpu.sync_copy(tgt_hbm.at[pl.ds(base, TPW)], raw_v)
        for b in range(NBUF):
            pltpu.sync_copy(table_hbm.at[pl.ds(0, 8)],
                            rows_v.at[b, pl.ds(0, 8)])

        def fire_gather(i, b):
            tok0 = i * L
            pltpu.async_copy(
                table_hbm.at[raw_v.at[pl.ds(tok0, CH)]],
                rows_v.at[b, pl.ds(1, CH)],
                gsem[b])
            pltpu.async_copy(
                table_hbm.at[raw_v.at[pl.ds(tok0 + CH, L - 1 - CH)]],
                rows_v.at[b, pl.ds(1 + CH, L - 1 - CH)],
                gsem[b])

        def wait_gather(b):
            pltpu.make_async_copy(
                table_hbm.at[raw_v.at[pl.ds(0, CH)]],
                rows_v.at[b, pl.ds(1, CH)],
                gsem[b]).wait()
            pltpu.make_async_copy(
                table_hbm.at[raw_v.at[pl.ds(CH, L - 1 - CH)]],
                rows_v.at[b, pl.ds(1 + CH, L - 1 - CH)],
                gsem[b]).wait()

        def start_store(i, b):
            pltpu.async_copy(
                rows_v.at[b], out_hbm.at[wid * SEQ_PER_W + i], ssem[b])

        def wait_store(b):
            pltpu.make_async_copy(
                rows_v.at[b], out_hbm.at[0], ssem[b]).wait()

        for b in range(NBUF - 1):
            fire_gather(b, b)

        def blk(g, carry):
            for b in range(NBUF):
                i = g * NBUF + b
                wait_gather(b)
                start_store(i, b)

                bb = (b + NBUF - 1) % NBUF

                @pl.when(i + NBUF - 1 < SEQ_PER_W)
                def _():
                    @pl.when(i >= 1)
                    def _():
                        wait_store(bb)
                    fire_gather(i + NBUF - 1, bb)
            return carry

        lax.fori_loop(0, SEQ_PER_W // NBUF, blk, 0)
        for b in range(NBUF):
            wait_store(b)

    return k(targets_flat, table_padded)


def kernel(targets, table):
    flat = targets.astype(jnp.int32).reshape(N_TOK)
    table_padded = _transpose_pad_table(table.T)
    out_padded = _embed_lookup(flat, table_padded)
    return out_padded[:, :, :D]


# speedup vs baseline: 1.5001x; 1.0254x over previous
import functools

import jax
import jax.numpy as jnp
from jax import lax
from jax.experimental import pallas as pl
from jax.experimental.pallas import tpu as pltpu
from jax.experimental.pallas import tpu_sc as plsc

VOCAB = 1000000
D = 64
DP = 128
B, L = 4096, 200
N_TOK = B * L
NW = 32
TPW = N_TOK // NW
SEQ_PER_W = TPW // L
SPB = 2
NBLK = SEQ_PER_W // SPB
CH = 128


VC = 16384


def _tp_body(in_ref, out_ref):
    out_ref[:, 0:D] = in_ref[...].T


def _transpose_pad_table(table_t):
    return pl.pallas_call(
        _tp_body,
        out_shape=jax.ShapeDtypeStruct((VOCAB, DP), jnp.float32),
        grid=(pl.cdiv(VOCAB, VC),),
        in_specs=[pl.BlockSpec((D, VC), lambda i: (0, i))],
        out_specs=pl.BlockSpec((VC, DP), lambda i: (i, 0)),
    )(table_t)


NBUF = 4


def _embed_lookup(targets_flat, table_padded):
    mesh = plsc.VectorSubcoreMesh(core_axis_name="c", subcore_axis_name="s")

    @functools.partial(
        pl.kernel,
        mesh=mesh,
        compiler_params=pltpu.CompilerParams(use_tc_tiling_on_sc=False),
        out_type=jax.ShapeDtypeStruct((B, L, DP), jnp.float32),
        scratch_types=[
            pltpu.VMEM((TPW,), jnp.int32),
            pltpu.VMEM((NBUF, L, DP), jnp.float32),
            [pltpu.SemaphoreType.DMA] * NBUF,
            [pltpu.SemaphoreType.DMA] * NBUF,
        ],
    )
    def k(tgt_hbm, table_hbm, out_hbm, raw_v, rows_v, gsem, ssem):
        wid = lax.axis_index("s") * 2 + lax.axis_index("c")
        base = wid * TPW
        pltpu.sync_copy(tgt_hbm.at[pl.ds(base, TPW)], raw_v)
        for b in range(NBUF):
            pltpu.sync_copy(table_hbm.at[pl.ds(0, 8)],
                            rows_v.at[b, pl.ds(0, 8)])

        def fire_gather(i, b):
            tok0 = i * L
            pltpu.async_copy(
                table_hbm.at[raw_v.at[pl.ds(tok0, CH)]],
                rows_v.at[b, pl.ds(1, CH)],
                gsem[b])
            pltpu.async_copy(
                table_hbm.at[raw_v.at[pl.ds(tok0 + CH, L - 1 - CH)]],
                rows_v.at[b, pl.ds(1 + CH, L - 1 - CH)],
                gsem[b])

        def wait_gather(b):
            pltpu.make_async_copy(
                table_hbm.at[raw_v.at[pl.ds(0, CH)]],
                rows_v.at[b, pl.ds(1, CH)],
                gsem[b]).wait()
            pltpu.make_async_copy(
                table_hbm.at[raw_v.at[pl.ds(CH, L - 1 - CH)]],
                rows_v.at[b, pl.ds(1 + CH, L - 1 - CH)],
                gsem[b]).wait()

        def start_store(i, b):
            pltpu.async_copy(
                rows_v.at[b], out_hbm.at[wid * SEQ_PER_W + i], ssem[b])

        def wait_store(b):
            pltpu.make_async_copy(
                rows_v.at[b], out_hbm.at[0], ssem[b]).wait()

        for b in range(NBUF - 1):
            fire_gather(b, b)

        def blk(g, carry):
            for b in range(NBUF):
                i = g * NBUF + b
                wait_gather(b)
                start_store(i, b)

                bb = (b + NBUF - 1) % NBUF

                @pl.when(i + NBUF - 1 < SEQ_PER_W)
                def _():
                    @pl.when(i >= 1)
                    def _():
                        wait_store(bb)
                    fire_gather(i + NBUF - 1, bb)
            return carry

        lax.fori_loop(0, SEQ_PER_W // NBUF, blk, 0)
        for b in range(NBUF):
            wait_store(b)

    return k(targets_flat, table_padded)


def kernel(targets, table):
    flat = targets.astype(jnp.int32).reshape(N_TOK)
    table_padded = _transpose_pad_table(table.T)
    out_padded = _embed_lookup(flat, table_padded)
    return out_padded[:, :, :D]


# speedup vs baseline: 1.8654x; 1.2435x over previous
import functools

import jax
import jax.numpy as jnp
from jax import lax
from jax.experimental import pallas as pl
from jax.experimental.pallas import tpu as pltpu
from jax.experimental.pallas import tpu_sc as plsc

VOCAB = 1000000
D = 64
DP = 128
B, L = 4096, 200
N_TOK = B * L
NW = 32
TPW = N_TOK // NW
SEQ_PER_W = TPW // L
SPB = 2
NBLK = SEQ_PER_W // SPB
CH = 128


VC = 16384


def _tp_body(in_ref, out_ref):
    out_ref[:, 0:D] = in_ref[...].T


def _transpose_pad_table(table_t):
    return pl.pallas_call(
        _tp_body,
        out_shape=jax.ShapeDtypeStruct((VOCAB, DP), jnp.float32),
        grid=(pl.cdiv(VOCAB, VC),),
        in_specs=[pl.BlockSpec((D, VC), lambda i: (0, i))],
        out_specs=pl.BlockSpec((VC, DP), lambda i: (i, 0)),
    )(table_t)


NBUF = 4


def _embed_lookup(targets_flat, table_padded):
    mesh = plsc.VectorSubcoreMesh(core_axis_name="c", subcore_axis_name="s")

    @functools.partial(
        pl.kernel,
        mesh=mesh,
        compiler_params=pltpu.CompilerParams(use_tc_tiling_on_sc=False),
        out_type=jax.ShapeDtypeStruct((B, L, DP), jnp.float32),
        scratch_types=[
            pltpu.VMEM((TPW,), jnp.int32),
            pltpu.VMEM((NBUF, L, D), jnp.float32),
            [pltpu.SemaphoreType.DMA] * NBUF,
            [pltpu.SemaphoreType.DMA] * NBUF,
        ],
    )
    def k(tgt_hbm, table_hbm, out_hbm, raw_v, rows_v, gsem, ssem):
        wid = lax.axis_index("s") * 2 + lax.axis_index("c")
        base = wid * TPW
        pltpu.sync_copy(tgt_hbm.at[pl.ds(base, TPW)], raw_v)
        for b in range(NBUF):
            pltpu.sync_copy(table_hbm.at[pl.ds(0, 8)],
                            rows_v.at[b, pl.ds(0, 8)])

        def fire_gather(i, b):
            tok0 = i * L
            pltpu.async_copy(
                table_hbm.at[raw_v.at[pl.ds(tok0, CH)]],
                rows_v.at[b, pl.ds(1, CH)],
                gsem[b])
            pltpu.async_copy(
                table_hbm.at[raw_v.at[pl.ds(tok0 + CH, L - 1 - CH)]],
                rows_v.at[b, pl.ds(1 + CH, L - 1 - CH)],
                gsem[b])

        def wait_gather(b):
            pltpu.make_async_copy(
                table_hbm.at[raw_v.at[pl.ds(0, CH)]],
                rows_v.at[b, pl.ds(1, CH)],
                gsem[b]).wait()
            pltpu.make_async_copy(
                table_hbm.at[raw_v.at[pl.ds(CH, L - 1 - CH)]],
                rows_v.at[b, pl.ds(1 + CH, L - 1 - CH)],
                gsem[b]).wait()

        def start_store(i, b):
            pltpu.async_copy(
                rows_v.at[b],
                out_hbm.at[wid * SEQ_PER_W + i, :, pl.ds(0, D)],
                ssem[b])

        def wait_store(b):
            pltpu.make_async_copy(
                rows_v.at[b], out_hbm.at[0, :, pl.ds(0, D)], ssem[b]).wait()

        for b in range(NBUF - 1):
            fire_gather(b, b)

        def blk(g, carry):
            for b in range(NBUF):
                i = g * NBUF + b
                wait_gather(b)
                start_store(i, b)

                bb = (b + NBUF - 1) % NBUF

                @pl.when(i + NBUF - 1 < SEQ_PER_W)
                def _():
                    @pl.when(i >= 1)
                    def _():
                        wait_store(bb)
                    fire_gather(i + NBUF - 1, bb)
            return carry

        lax.fori_loop(0, SEQ_PER_W // NBUF, blk, 0)
        for b in range(NBUF):
            wait_store(b)

    return k(targets_flat, table_padded)


def kernel(targets, table):
    flat = (targets.astype(jnp.int32) * 2).reshape(N_TOK)
    table_padded = _transpose_pad_table(table.T)
    out_padded = _embed_lookup(flat, table_padded.reshape(2 * VOCAB, D))
    return out_padded[:, :, :D]
